# baseline (speedup 1.0000x reference)
import functools

import jax
import jax.numpy as jnp
import numpy as np
from jax.experimental import pallas as pl
from jax.experimental.pallas import tpu as pltpu

_ALPHA = 0.98
_DELTA = 2.0
_R = 0.5
_S = 0.025
_EPS = 1e-6
_A = 1.0 - _S


def _pcen_kernel(x_ref, l_ref, apow_ref, o_ref, carry_ref, *, t_total, w):
    t = pl.program_id(1)
    bb, c, _ = x_ref.shape
    rows = bb * c
    xr = x_ref[...].reshape(rows, w)
    lanes = jax.lax.broadcasted_iota(jnp.int32, (rows, w), 1)
    xb = jnp.where(lanes < (t_total - t * w), xr, 0.0)

    @pl.when(t == 0)
    def _():
        carry_ref[...] = jnp.broadcast_to(xb[:, 0:1], carry_ref.shape)

    sm = (
        jnp.dot(xb, l_ref[...], preferred_element_type=jnp.float32)
        + carry_ref[...] * apow_ref[...]
    )
    carry_ref[...] = jnp.broadcast_to(sm[:, w - 1 : w], carry_ref.shape)

    inv_pow = jnp.exp2(jnp.log2(sm + _EPS) * (-_ALPHA))
    out = jnp.sqrt(xb * inv_pow + _DELTA) - np.float32(np.sqrt(_DELTA))
    o_ref[...] = out.reshape(bb, c, w)


def _build_consts(w):
    k = np.arange(w)[:, None].astype(np.float64)
    j = np.arange(w)[None, :].astype(np.float64)
    l_mat = np.where(j >= k, _S * _A ** (j - k), 0.0).astype(np.float32)
    apow = (_A ** (np.arange(w, dtype=np.float64) + 1.0)).astype(np.float32)
    return l_mat, apow.reshape(1, w)


@jax.jit
def kernel(x):
    b, c, t_total = x.shape

    w = 256
    bb = 16
    n_chunks = pl.cdiv(t_total, w)
    n_row_tiles = pl.cdiv(b, bb)

    l_mat, apow = _build_consts(w)

    return pl.pallas_call(
        functools.partial(_pcen_kernel, t_total=t_total, w=w),
        out_shape=jax.ShapeDtypeStruct((b, c, t_total), jnp.float32),
        grid=(n_row_tiles, n_chunks),
        in_specs=[
            pl.BlockSpec((bb, c, w), lambda i, t: (i, 0, t)),
            pl.BlockSpec((w, w), lambda i, t: (0, 0)),
            pl.BlockSpec((1, w), lambda i, t: (0, 0)),
        ],
        out_specs=pl.BlockSpec((bb, c, w), lambda i, t: (i, 0, t)),
        scratch_shapes=[pltpu.VMEM((bb * c, w), jnp.float32)],
        compiler_params=pltpu.CompilerParams(
            dimension_semantics=("parallel", "arbitrary"),
        ),
        name="pcen",
    )(x, jnp.asarray(l_mat), jnp.asarray(apow))


# speedup vs baseline: 1.3922x; 1.3922x over previous
import functools

import jax
import jax.numpy as jnp
import numpy as np
from jax.experimental import pallas as pl
from jax.experimental.pallas import tpu as pltpu

_ALPHA = 0.98
_DELTA = 2.0
_R = 0.5
_S = 0.025
_EPS = 1e-6
_A = 1.0 - _S


def _pcen_kernel(x_ref, l_ref, apow_ref, o_ref, carry_ref, *, t_total, wt):
    t = pl.program_id(1)
    bb, _, c = x_ref.shape
    sub = jax.lax.broadcasted_iota(jnp.int32, (wt, c), 0)
    mask = sub < (t_total - t * wt)
    l2 = l_ref[...]
    ap = apow_ref[...]
    sqrt_delta = np.float32(np.sqrt(_DELTA))

    for b in range(bb):
        xb = jnp.where(mask, x_ref[b], 0.0)

        @pl.when(t == 0)
        def _(b=b, xb=xb):
            carry_ref[b, 0:1, :] = xb[0:1, :]

        cb = jnp.broadcast_to(carry_ref[b, 0:1, :], (wt, c))
        sm = jnp.dot(l2, xb, preferred_element_type=jnp.float32) + ap * cb
        carry_ref[b, 0:1, :] = sm[wt - 1 : wt, :]

        inv_pow = jnp.exp2(jnp.log2(sm + _EPS) * (-_ALPHA))
        u = xb * inv_pow + _DELTA
        o_ref[b] = jax.lax.rsqrt(u) * u - sqrt_delta


def _build_consts(wt, c):
    j = np.arange(wt)[:, None].astype(np.float64)
    k = np.arange(wt)[None, :].astype(np.float64)
    l_mat = np.where(j >= k, _S * _A ** (j - k), 0.0).astype(np.float32)
    apow = np.broadcast_to(
        (_A ** (np.arange(wt, dtype=np.float64) + 1.0)).astype(np.float32)[:, None],
        (wt, c),
    ).copy()
    return l_mat, apow


@jax.jit
def kernel(x):
    b, c, t_total = x.shape
    xt = jnp.transpose(x, (0, 2, 1))

    wt = 256
    bb = 8
    n_chunks = pl.cdiv(t_total, wt)
    n_b_tiles = pl.cdiv(b, bb)

    l_mat, apow = _build_consts(wt, c)

    out = pl.pallas_call(
        functools.partial(_pcen_kernel, t_total=t_total, wt=wt),
        out_shape=jax.ShapeDtypeStruct((b, t_total, c), jnp.float32),
        grid=(n_b_tiles, n_chunks),
        in_specs=[
            pl.BlockSpec((bb, wt, c), lambda i, t: (i, t, 0)),
            pl.BlockSpec((wt, wt), lambda i, t: (0, 0)),
            pl.BlockSpec((wt, c), lambda i, t: (0, 0)),
        ],
        out_specs=pl.BlockSpec((bb, wt, c), lambda i, t: (i, t, 0)),
        scratch_shapes=[pltpu.VMEM((bb, 8, c), jnp.float32)],
        compiler_params=pltpu.CompilerParams(
            dimension_semantics=("parallel", "arbitrary"),
        ),
        name="pcen",
    )(xt, jnp.asarray(l_mat), jnp.asarray(apow))

    return jnp.transpose(out, (0, 2, 1))


# speedup vs baseline: 2.2563x; 1.6207x over previous
import functools

import jax
import jax.numpy as jnp
import numpy as np
from jax.experimental import pallas as pl
from jax.experimental.pallas import tpu as pltpu

_ALPHA = 0.98
_DELTA = 2.0
_R = 0.5
_S = 0.025
_EPS = 1e-6
_A = 1.0 - _S


def _pcen_kernel(x_ref, l_ref, apow_ref, o_ref, carry_ref, *, t_total, wt):
    t = pl.program_id(1)
    bb, _, c = x_ref.shape
    n = bb * c
    xw = jnp.concatenate([x_ref[b] for b in range(bb)], axis=1)
    sub = jax.lax.broadcasted_iota(jnp.int32, (wt, n), 0)
    xb = jnp.where(sub < (t_total - t * wt), xw, 0.0)

    @pl.when(t == 0)
    def _():
        carry_ref[...] = xb[0:1, :]

    ap = jnp.concatenate([apow_ref[...]] * bb, axis=1)
    cb = jnp.broadcast_to(carry_ref[...], (wt, n))
    sm = jnp.dot(l_ref[...], xb, preferred_element_type=jnp.float32) + ap * cb
    carry_ref[...] = sm[wt - 1 : wt, :]

    inv_pow = jnp.exp2(jnp.log2(sm + _EPS) * (-_ALPHA))
    u = xb * inv_pow + _DELTA
    out = jax.lax.rsqrt(u) * u - np.float32(np.sqrt(_DELTA))
    for b in range(bb):
        o_ref[b] = out[:, b * c : (b + 1) * c]


def _build_consts(wt, c):
    j = np.arange(wt)[:, None].astype(np.float64)
    k = np.arange(wt)[None, :].astype(np.float64)
    l_mat = np.where(j >= k, _S * _A ** (j - k), 0.0).astype(np.float32)
    apow = np.broadcast_to(
        (_A ** (np.arange(wt, dtype=np.float64) + 1.0)).astype(np.float32)[:, None],
        (wt, c),
    ).copy()
    return l_mat, apow


@jax.jit
def kernel(x):
    b, c, t_total = x.shape
    xt = jnp.transpose(x, (0, 2, 1))

    wt = 256
    bb = 8
    n_chunks = pl.cdiv(t_total, wt)
    n_b_tiles = pl.cdiv(b, bb)

    l_mat, apow = _build_consts(wt, c)

    out = pl.pallas_call(
        functools.partial(_pcen_kernel, t_total=t_total, wt=wt),
        out_shape=jax.ShapeDtypeStruct((b, t_total, c), jnp.float32),
        grid=(n_b_tiles, n_chunks),
        in_specs=[
            pl.BlockSpec((bb, wt, c), lambda i, t: (i, t, 0)),
            pl.BlockSpec((wt, wt), lambda i, t: (0, 0)),
            pl.BlockSpec((wt, c), lambda i, t: (0, 0)),
        ],
        out_specs=pl.BlockSpec((bb, wt, c), lambda i, t: (i, t, 0)),
        scratch_shapes=[pltpu.VMEM((1, bb * c), jnp.float32)],
        compiler_params=pltpu.CompilerParams(
            dimension_semantics=("parallel", "arbitrary"),
        ),
        name="pcen",
    )(xt, jnp.asarray(l_mat), jnp.asarray(apow))

    return jnp.transpose(out, (0, 2, 1))


# speedup vs baseline: 2.9104x; 1.2899x over previous
import functools

import jax
import jax.numpy as jnp
import numpy as np
from jax.experimental import pallas as pl
from jax.experimental.pallas import tpu as pltpu

_ALPHA = 0.98
_DELTA = 2.0
_R = 0.5
_S = 0.025
_EPS = 1e-6
_A = 1.0 - _S


def _pcen_kernel(x_ref, l_ref, apow_ref, o_ref, carry_ref, *, t_total, wt):
    t = pl.program_id(1)
    bb, _, c = x_ref.shape
    n = bb * c
    xw = jnp.concatenate([x_ref[b] for b in range(bb)], axis=1)
    sub = jax.lax.broadcasted_iota(jnp.int32, (wt, n), 0)
    xb = jnp.where(sub < (t_total - t * wt), xw, 0.0)

    @pl.when(t == 0)
    def _():
        carry_ref[...] = xb[0:1, :]

    ap = jnp.concatenate([apow_ref[...]] * bb, axis=1)
    cb = jnp.broadcast_to(carry_ref[...], (wt, n))
    sm = jnp.dot(l_ref[...], xb, preferred_element_type=jnp.float32) + ap * cb
    carry_ref[...] = sm[wt - 1 : wt, :]

    inv_pow = jnp.exp2(jnp.log2(sm + _EPS) * (-_ALPHA))
    u = xb * inv_pow + _DELTA
    out = jax.lax.rsqrt(u) * u - np.float32(np.sqrt(_DELTA))
    for b in range(bb):
        o_ref[b] = out[:, b * c : (b + 1) * c]


def _build_consts(wt, c):
    j = np.arange(wt)[:, None].astype(np.float64)
    k = np.arange(wt)[None, :].astype(np.float64)
    l_mat = np.where(j >= k, _S * _A ** (j - k), 0.0).astype(np.float32)
    apow = np.broadcast_to(
        (_A ** (np.arange(wt, dtype=np.float64) + 1.0)).astype(np.float32)[:, None],
        (wt, c),
    ).copy()
    return l_mat, apow


@jax.jit
def kernel(x):
    b, c, t_total = x.shape
    xt = jnp.transpose(x, (0, 2, 1))

    wt = 256
    bb = 16
    n_chunks = pl.cdiv(t_total, wt)
    n_b_tiles = pl.cdiv(b, bb)

    l_mat, apow = _build_consts(wt, c)

    out = pl.pallas_call(
        functools.partial(_pcen_kernel, t_total=t_total, wt=wt),
        out_shape=jax.ShapeDtypeStruct((b, t_total, c), jnp.float32),
        grid=(n_b_tiles, n_chunks),
        in_specs=[
            pl.BlockSpec((bb, wt, c), lambda i, t: (i, t, 0)),
            pl.BlockSpec((wt, wt), lambda i, t: (0, 0)),
            pl.BlockSpec((wt, c), lambda i, t: (0, 0)),
        ],
        out_specs=pl.BlockSpec((bb, wt, c), lambda i, t: (i, t, 0)),
        scratch_shapes=[pltpu.VMEM((1, bb * c), jnp.float32)],
        compiler_params=pltpu.CompilerParams(
            dimension_semantics=("parallel", "arbitrary"),
        ),
        name="pcen",
    )(xt, jnp.asarray(l_mat), jnp.asarray(apow))

    return jnp.transpose(out, (0, 2, 1))


# speedup vs baseline: 3.4022x; 1.1690x over previous
import functools

import jax
import jax.numpy as jnp
import numpy as np
from jax.experimental import pallas as pl
from jax.experimental.pallas import tpu as pltpu

_ALPHA = 0.98
_DELTA = 2.0
_R = 0.5
_S = 0.025
_EPS = 1e-6
_A = 1.0 - _S


def _pcen_kernel(x_ref, l_ref, apow_ref, o_ref, carry_ref, *, t_total, wt):
    t = pl.program_id(1)
    bb, _, c = x_ref.shape
    n = bb * c
    xw = jnp.concatenate([x_ref[b] for b in range(bb)], axis=1)
    sub = jax.lax.broadcasted_iota(jnp.int32, (wt, n), 0)
    xb = jnp.where(sub < (t_total - t * wt), xw, 0.0)

    @pl.when(t == 0)
    def _():
        carry_ref[...] = xb[0:1, :]

    ap = jnp.concatenate([apow_ref[...]] * bb, axis=1)
    cb = jnp.broadcast_to(carry_ref[...], (wt, n))
    sm = jnp.dot(l_ref[...], xb, preferred_element_type=jnp.float32) + ap * cb
    carry_ref[...] = sm[wt - 1 : wt, :]

    inv_pow = jnp.exp2(jnp.log2(sm + _EPS) * (-_ALPHA))
    u = xb * inv_pow + _DELTA
    out = jax.lax.rsqrt(u) * u - np.float32(np.sqrt(_DELTA))
    for b in range(bb):
        o_ref[b] = out[:, b * c : (b + 1) * c]


def _build_consts(wt, c):
    j = np.arange(wt)[:, None].astype(np.float64)
    k = np.arange(wt)[None, :].astype(np.float64)
    l_mat = np.where(j >= k, _S * _A ** (j - k), 0.0).astype(np.float32)
    apow = np.broadcast_to(
        (_A ** (np.arange(wt, dtype=np.float64) + 1.0)).astype(np.float32)[:, None],
        (wt, c),
    ).copy()
    return l_mat, apow


@jax.jit
def kernel(x):
    b, c, t_total = x.shape
    xt = jnp.transpose(x, (0, 2, 1))

    wt = 256
    bb = 32
    n_chunks = pl.cdiv(t_total, wt)
    n_b_tiles = pl.cdiv(b, bb)

    l_mat, apow = _build_consts(wt, c)

    out = pl.pallas_call(
        functools.partial(_pcen_kernel, t_total=t_total, wt=wt),
        out_shape=jax.ShapeDtypeStruct((b, t_total, c), jnp.float32),
        grid=(n_b_tiles, n_chunks),
        in_specs=[
            pl.BlockSpec((bb, wt, c), lambda i, t: (i, t, 0)),
            pl.BlockSpec((wt, wt), lambda i, t: (0, 0)),
            pl.BlockSpec((wt, c), lambda i, t: (0, 0)),
        ],
        out_specs=pl.BlockSpec((bb, wt, c), lambda i, t: (i, t, 0)),
        scratch_shapes=[pltpu.VMEM((1, bb * c), jnp.float32)],
        compiler_params=pltpu.CompilerParams(
            dimension_semantics=("parallel", "arbitrary"),
        ),
        name="pcen",
    )(xt, jnp.asarray(l_mat), jnp.asarray(apow))

    return jnp.transpose(out, (0, 2, 1))


# speedup vs baseline: 3.6558x; 1.0745x over previous
import functools

import jax
import jax.numpy as jnp
import numpy as np
from jax.experimental import pallas as pl
from jax.experimental.pallas import tpu as pltpu

_ALPHA = 0.98
_DELTA = 2.0
_R = 0.5
_S = 0.025
_EPS = 1e-6
_A = 1.0 - _S


def _pcen_kernel(x_ref, l_ref, apow_ref, o_ref, carry_ref, *, t_total, wt):
    t = pl.program_id(1)
    bb, _, c = x_ref.shape
    n = bb * c
    xw = jnp.concatenate([x_ref[b] for b in range(bb)], axis=1)
    sub = jax.lax.broadcasted_iota(jnp.int32, (wt, n), 0)
    xb = jnp.where(sub < (t_total - t * wt), xw, 0.0)

    @pl.when(t == 0)
    def _():
        carry_ref[...] = xb[0:1, :]

    ap = jnp.concatenate([apow_ref[...]] * bb, axis=1)
    cb = jnp.broadcast_to(carry_ref[...], (wt, n))
    sm = jnp.dot(l_ref[...], xb, preferred_element_type=jnp.float32) + ap * cb
    carry_ref[...] = sm[wt - 1 : wt, :]

    inv_pow = jnp.exp2(jnp.log2(sm + _EPS) * (-_ALPHA))
    u = xb * inv_pow + _DELTA
    out = jax.lax.rsqrt(u) * u - np.float32(np.sqrt(_DELTA))
    for b in range(bb):
        o_ref[b] = out[:, b * c : (b + 1) * c]


def _build_consts(wt, c):
    j = np.arange(wt)[:, None].astype(np.float64)
    k = np.arange(wt)[None, :].astype(np.float64)
    l_mat = np.where(j >= k, _S * _A ** (j - k), 0.0).astype(np.float32)
    apow = np.broadcast_to(
        (_A ** (np.arange(wt, dtype=np.float64) + 1.0)).astype(np.float32)[:, None],
        (wt, c),
    ).copy()
    return l_mat, apow


@jax.jit
def kernel(x):
    b, c, t_total = x.shape
    xt = jnp.transpose(x, (0, 2, 1))

    wt = 256
    bb = 64
    n_chunks = pl.cdiv(t_total, wt)
    n_b_tiles = pl.cdiv(b, bb)

    l_mat, apow = _build_consts(wt, c)

    out = pl.pallas_call(
        functools.partial(_pcen_kernel, t_total=t_total, wt=wt),
        out_shape=jax.ShapeDtypeStruct((b, t_total, c), jnp.float32),
        grid=(n_b_tiles, n_chunks),
        in_specs=[
            pl.BlockSpec((bb, wt, c), lambda i, t: (i, t, 0)),
            pl.BlockSpec((wt, wt), lambda i, t: (0, 0)),
            pl.BlockSpec((wt, c), lambda i, t: (0, 0)),
        ],
        out_specs=pl.BlockSpec((bb, wt, c), lambda i, t: (i, t, 0)),
        scratch_shapes=[pltpu.VMEM((1, bb * c), jnp.float32)],
        compiler_params=pltpu.CompilerParams(
            dimension_semantics=("parallel", "arbitrary"),
        ),
        name="pcen",
    )(xt, jnp.asarray(l_mat), jnp.asarray(apow))

    return jnp.transpose(out, (0, 2, 1))
